# baseline (speedup 1.0000x reference)
import functools

import jax
import jax.numpy as jnp
from jax.experimental import pallas as pl
from jax.experimental.pallas import tpu as pltpu

NUM_CLASSES = 1000
CLASS_PAD = 1024
M_BLOCK = 448


def _onehot_matmul_kernel(x_ref, lbl_ref, out_ref, xb_ref, p_ref, inv_ref):
    i = pl.program_id(0)
    nsteps = pl.num_programs(0)

    @pl.when(i == 0)
    def _build_p():
        lbl = lbl_ref[...]
        classes = jax.lax.broadcasted_iota(jnp.int32, (1, CLASS_PAD), 1)
        onehot = lbl == classes
        p_ref[...] = onehot.astype(jnp.bfloat16)
        counts = jnp.sum(onehot.astype(jnp.float32), axis=0, keepdims=True)
        inv_ref[...] = 1.0 / jnp.maximum(counts, 1.0)

    @pl.when(i < nsteps - 1)
    def _convert():
        xb_ref[i % 2] = x_ref[...].astype(jnp.bfloat16)

    @pl.when(i > 0)
    def _matmul():
        acc = jnp.dot(
            xb_ref[(i - 1) % 2], p_ref[...], preferred_element_type=jnp.float32
        )
        out_ref[...] = (acc * inv_ref[...])[:, :NUM_CLASSES]


@jax.jit
def _scatter_mean(x2, lbl2):
    m, n = x2.shape
    nblk = m // M_BLOCK

    out = pl.pallas_call(
        _onehot_matmul_kernel,
        grid=(nblk + 1,),
        in_specs=[
            pl.BlockSpec((M_BLOCK, n), lambda i: (jnp.minimum(i, nblk - 1), 0)),
            pl.BlockSpec((n, 1), lambda i: (0, 0)),
        ],
        out_specs=pl.BlockSpec(
            (M_BLOCK, NUM_CLASSES), lambda i: (jnp.maximum(i - 1, 0), 0)
        ),
        out_shape=jax.ShapeDtypeStruct((m, NUM_CLASSES), jnp.float32),
        scratch_shapes=[
            pltpu.VMEM((2, M_BLOCK, n), jnp.bfloat16),
            pltpu.VMEM((n, CLASS_PAD), jnp.bfloat16),
            pltpu.VMEM((1, CLASS_PAD), jnp.float32),
        ],
        compiler_params=pltpu.CompilerParams(
            dimension_semantics=("arbitrary",),
        ),
    )(x2, lbl2)
    return out


def kernel(x, label):
    n, c, h, w = x.shape
    f = c * h * w
    x2 = x.transpose(2, 3, 1, 0).reshape(f, n)
    lbl2 = label.astype(jnp.int32).reshape(n, 1)
    out2 = _scatter_mean(x2, lbl2)
    return out2.reshape(h, w, c, NUM_CLASSES).transpose(3, 2, 0, 1)


# speedup vs baseline: 1.0847x; 1.0847x over previous
import functools

import jax
import jax.numpy as jnp
from jax.experimental import pallas as pl
from jax.experimental.pallas import tpu as pltpu

NUM_CLASSES = 1000
CLASS_PAD = 1024
M_BLOCK = 896


def _onehot_matmul_kernel(x_ref, lbl_ref, out_ref, p_ref, inv_ref):
    i = pl.program_id(0)

    @pl.when(i == 0)
    def _build_p():
        classes = jax.lax.broadcasted_iota(jnp.int32, (1, CLASS_PAD), 1)
        counts = jnp.zeros((1, CLASS_PAD), jnp.float32)
        n = lbl_ref.shape[0]
        kc = n // 4
        for k in range(4):
            sl = pl.ds(k * kc, kc)
            onehot = (lbl_ref[sl, :] == classes).astype(jnp.float32)
            counts = counts + jnp.sum(onehot, axis=0, keepdims=True)
            p_ref[sl, :] = onehot.astype(jnp.bfloat16)
        inv_ref[...] = 1.0 / jnp.maximum(counts, 1.0)

    half = M_BLOCK // 2
    inv = inv_ref[...]
    for h in range(2):
        sl = pl.ds(h * half, half)
        xb = x_ref[sl, :].astype(jnp.bfloat16)
        acc = jnp.dot(xb, p_ref[...], preferred_element_type=jnp.float32)
        out_ref[sl, :] = (acc * inv)[:, :NUM_CLASSES]


@jax.jit
def _scatter_mean(x2, lbl2):
    m, n = x2.shape
    nblk = m // M_BLOCK

    out = pl.pallas_call(
        _onehot_matmul_kernel,
        grid=(nblk,),
        in_specs=[
            pl.BlockSpec((M_BLOCK, n), lambda i: (i, 0)),
            pl.BlockSpec((n, 1), lambda i: (0, 0)),
        ],
        out_specs=pl.BlockSpec((M_BLOCK, NUM_CLASSES), lambda i: (i, 0)),
        out_shape=jax.ShapeDtypeStruct((m, NUM_CLASSES), jnp.float32),
        scratch_shapes=[
            pltpu.VMEM((n, CLASS_PAD), jnp.bfloat16),
            pltpu.VMEM((1, CLASS_PAD), jnp.float32),
        ],
        compiler_params=pltpu.CompilerParams(
            dimension_semantics=("arbitrary",),
        ),
    )(x2, lbl2)
    return out


def kernel(x, label):
    n, c, h, w = x.shape
    f = c * h * w
    x2 = x.transpose(2, 3, 1, 0).reshape(f, n)
    lbl2 = label.astype(jnp.int32).reshape(n, 1)
    out2 = _scatter_mean(x2, lbl2)
    return out2.reshape(h, w, c, NUM_CLASSES).transpose(3, 2, 0, 1)
